# baseline (speedup 1.0000x reference)
import functools

import jax
import jax.numpy as jnp
from jax import lax
from jax.experimental import pallas as pl
from jax.experimental.pallas import tpu as pltpu
from jax.experimental.pallas import tpu_sc as plsc

N = 4096
NC = 2
NS = 16
NW = NC * NS
L = 16
NSC = 1536
ROWS = NSC // NW
CHUNKS = N // L
TC_BR = 256


def _sc_body(e_hbm, t_hbm, est_hbm, out_hbm, e_v, t_v, est_v, k_v, part_v):
    cid = lax.axis_index("c")
    sid = lax.axis_index("s")
    wid = sid * NC + cid

    pltpu.sync_copy(e_hbm, e_v)
    pltpu.sync_copy(t_hbm, t_v)
    pltpu.sync_copy(est_hbm, est_v)

    def _key_chunk(i, carry):
        sl = pl.ds(i * L, L)
        kb = lax.bitcast_convert_type(jnp.exp(t_v[sl]), jnp.int32)
        k_v[sl] = kb + (1 - e_v[sl])
        return carry

    lax.fori_loop(0, CHUNKS, _key_chunk, 0)

    zeros = jnp.zeros((L,), jnp.float32)
    ones = jnp.ones((L,), jnp.float32)

    part_v[0, :] = zeros
    part_v[1, :] = zeros
    part_v[2, :] = zeros

    def _row_chunk(rc, carry0):
        base = wid * ROWS + rc * L
        rowk = k_v[pl.ds(base, L)]
        rowe = e_v[pl.ds(base, L)]
        rowest = est_v[pl.ds(base, L)]
        for r in range(L):
            ka_s = rowk[r]
            esta_s = rowest[r]
            ea_s = rowe[r]

            @pl.when(ea_s > 0)
            def _do_row(ka_s=ka_s, esta_s=esta_s):
                ka = jnp.full((L,), ka_s, jnp.int32)
                esta = jnp.full((L,), esta_s, jnp.float32)

                def _chunk(c, carry):
                    acc_t, acc_c, acc_e = carry
                    off = c * L
                    kb = k_v[pl.ds(off, L)]
                    estb = est_v[pl.ds(off, L)]
                    m1 = kb > ka
                    m_con = estb < esta
                    m_tie = jnp.abs(estb - esta) <= 1e-8
                    acc_t = acc_t + jnp.where(m1, ones, zeros)
                    acc_c = acc_c + jnp.where(m1 & m_con, ones, zeros)
                    acc_e = acc_e + jnp.where(m1 & m_tie, ones, zeros)
                    return acc_t, acc_c, acc_e

                acc_t, acc_c, acc_e = lax.fori_loop(
                    0, CHUNKS, _chunk, (zeros, zeros, zeros), unroll=8)
                part_v[0, :] = part_v[0, :] + acc_t
                part_v[1, :] = part_v[1, :] + acc_c
                part_v[2, :] = part_v[2, :] + acc_e

        return carry0

    lax.fori_loop(0, ROWS // L, _row_chunk, 0)
    pltpu.sync_copy(part_v, out_hbm.at[wid])


@jax.jit
def _sc_counts(e, t, est):
    mesh = plsc.VectorSubcoreMesh(core_axis_name="c", subcore_axis_name="s")
    f = functools.partial(
        pl.kernel,
        mesh=mesh,
        out_type=jax.ShapeDtypeStruct((NW, 3, L), jnp.float32),
        scratch_types=[
            pltpu.VMEM((N,), jnp.int32),
            pltpu.VMEM((N,), jnp.float32),
            pltpu.VMEM((N,), jnp.float32),
            pltpu.VMEM((N,), jnp.int32),
            pltpu.VMEM((3, L), jnp.float32),
        ],
    )(_sc_body)
    return f(e, t, est)


def _tc_body(tr_ref, er_ref, estr_ref, tc_ref, ec_ref, estc_ref, out_ref):
    i = pl.program_id(0)
    ka = lax.bitcast_convert_type(
        jnp.exp(jnp.reshape(tr_ref[...], (TC_BR, 1))), jnp.int32)
    ea = jnp.reshape(er_ref[...], (TC_BR, 1))
    esta = jnp.reshape(estr_ref[...], (TC_BR, 1))
    kb = lax.bitcast_convert_type(jnp.exp(tc_ref[...]), jnp.int32)
    kb = kb + (1 - ec_ref[...])
    estb = estc_ref[...]
    comp = (ea > 0) & (kb > ka)
    comp_f = jnp.where(comp, 1.0, 0.0)
    con_f = jnp.where(estb < esta, comp_f, 0.0)
    tie_f = jnp.where(jnp.abs(estb - esta) <= 1e-8, comp_f, 0.0)

    @pl.when(i == 0)
    def _init():
        out_ref[0] = 0.0
        out_ref[1] = 0.0
        out_ref[2] = 0.0

    ones_col = jnp.ones((N, 1), jnp.float32)
    out_ref[0] += jnp.sum(jnp.dot(comp_f, ones_col,
                                  preferred_element_type=jnp.float32))
    out_ref[1] += jnp.sum(jnp.dot(con_f, ones_col,
                                  preferred_element_type=jnp.float32))
    out_ref[2] += jnp.sum(jnp.dot(tie_f, ones_col,
                                  preferred_element_type=jnp.float32))


@jax.jit
def _tc_counts(t, e, est):
    grid = (N - NSC) // TC_BR
    off = NSC // TC_BR
    row_spec = pl.BlockSpec((1, TC_BR), lambda i: (0, i + off))
    col_spec = pl.BlockSpec((1, N), lambda i: (0, 0))
    return pl.pallas_call(
        _tc_body,
        grid=(grid,),
        in_specs=[row_spec, row_spec, row_spec, col_spec, col_spec, col_spec],
        out_specs=pl.BlockSpec(memory_space=pltpu.SMEM),
        out_shape=jax.ShapeDtypeStruct((3,), jnp.float32),
    )(t, e, est, t, e, est)


def kernel(event_indicator, event_time, estimate):
    e = jnp.reshape(event_indicator, (-1,)).astype(jnp.int32)
    t = jnp.reshape(event_time, (-1,)).astype(jnp.float32)
    est = jnp.reshape(estimate, (-1,)).astype(jnp.float32)
    parts = _sc_counts(e, t, est)
    tc = _tc_counts(t.reshape(1, -1), e.reshape(1, -1), est.reshape(1, -1))
    total = jnp.sum(parts[:, 0, :]) + tc[0]
    con = jnp.sum(parts[:, 1, :]) + tc[1]
    tie = jnp.sum(parts[:, 2, :]) + tc[2]
    disc = total - con - tie
    loss = (disc + 0.5 * tie) / (disc + con + tie + 1e-07)
    return 1.0 - loss


# speedup vs baseline: 1.0863x; 1.0863x over previous
import functools

import jax
import jax.numpy as jnp
from jax import lax
from jax.experimental import pallas as pl
from jax.experimental.pallas import tpu as pltpu
from jax.experimental.pallas import tpu_sc as plsc

N = 4096
NC = 2
NS = 16
NW = NC * NS
L = 16
NSC = 1280
ROWS = NSC // NW
ROWS_TAIL = ROWS % L
CHUNKS = N // L
TC_BR = 256


def _sc_body(e_hbm, t_hbm, est_hbm, out_hbm, e_v, t_v, est_v, k_v, part_v):
    cid = lax.axis_index("c")
    sid = lax.axis_index("s")
    wid = sid * NC + cid

    pltpu.sync_copy(e_hbm, e_v)
    pltpu.sync_copy(t_hbm, t_v)
    pltpu.sync_copy(est_hbm, est_v)

    def _key_chunk(i, carry):
        sl = pl.ds(i * L, L)
        kb = lax.bitcast_convert_type(jnp.exp(t_v[sl]), jnp.int32)
        k_v[sl] = kb + (1 - e_v[sl])
        return carry

    lax.fori_loop(0, CHUNKS, _key_chunk, 0)

    zeros = jnp.zeros((L,), jnp.float32)
    ones = jnp.ones((L,), jnp.float32)

    part_v[0, :] = zeros
    part_v[1, :] = zeros
    part_v[2, :] = zeros

    def _process_rows(base, nrows):
        rowk = k_v[pl.ds(base, L)]
        rowe = e_v[pl.ds(base, L)]
        rowest = est_v[pl.ds(base, L)]
        for r in range(nrows):
            ka_s = rowk[r]
            esta_s = rowest[r]
            ea_s = rowe[r]

            @pl.when(ea_s > 0)
            def _do_row(ka_s=ka_s, esta_s=esta_s):
                ka = jnp.full((L,), ka_s, jnp.int32)
                esta = jnp.full((L,), esta_s, jnp.float32)

                def _chunk(c, carry):
                    acc_t, acc_c, acc_e = carry
                    off = c * L
                    kb = k_v[pl.ds(off, L)]
                    estb = est_v[pl.ds(off, L)]
                    m1 = kb > ka
                    m_con = estb < esta
                    m_tie = jnp.abs(estb - esta) <= 1e-8
                    acc_t = acc_t + jnp.where(m1, ones, zeros)
                    acc_c = acc_c + jnp.where(m1 & m_con, ones, zeros)
                    acc_e = acc_e + jnp.where(m1 & m_tie, ones, zeros)
                    return acc_t, acc_c, acc_e

                acc_t, acc_c, acc_e = lax.fori_loop(
                    0, CHUNKS, _chunk, (zeros, zeros, zeros), unroll=4)
                part_v[0, :] = part_v[0, :] + acc_t
                part_v[1, :] = part_v[1, :] + acc_c
                part_v[2, :] = part_v[2, :] + acc_e

    def _row_chunk(rc, carry0):
        _process_rows(wid * ROWS + rc * L, L)
        return carry0

    lax.fori_loop(0, ROWS // L, _row_chunk, 0)
    if ROWS_TAIL:
        _process_rows(wid * ROWS + (ROWS // L) * L, ROWS_TAIL)
    pltpu.sync_copy(part_v, out_hbm.at[wid])


@jax.jit
def _sc_counts(e, t, est):
    mesh = plsc.VectorSubcoreMesh(core_axis_name="c", subcore_axis_name="s")
    f = functools.partial(
        pl.kernel,
        mesh=mesh,
        out_type=jax.ShapeDtypeStruct((NW, 3, L), jnp.float32),
        scratch_types=[
            pltpu.VMEM((N,), jnp.int32),
            pltpu.VMEM((N,), jnp.float32),
            pltpu.VMEM((N,), jnp.float32),
            pltpu.VMEM((N,), jnp.int32),
            pltpu.VMEM((3, L), jnp.float32),
        ],
    )(_sc_body)
    return f(e, t, est)


def _tc_body(tr_ref, er_ref, estr_ref, tc_ref, ec_ref, estc_ref, out_ref):
    i = pl.program_id(0)
    ka = lax.bitcast_convert_type(
        jnp.exp(jnp.reshape(tr_ref[...], (TC_BR, 1))), jnp.int32)
    ea = jnp.reshape(er_ref[...], (TC_BR, 1))
    esta = jnp.reshape(estr_ref[...], (TC_BR, 1))
    kb = lax.bitcast_convert_type(jnp.exp(tc_ref[...]), jnp.int32)
    kb = kb + (1 - ec_ref[...])
    estb = estc_ref[...]
    comp = (ea > 0) & (kb > ka)
    comp_f = jnp.where(comp, 1.0, 0.0)
    con_f = jnp.where(estb < esta, comp_f, 0.0)
    tie_f = jnp.where(jnp.abs(estb - esta) <= 1e-8, comp_f, 0.0)

    @pl.when(i == 0)
    def _init():
        out_ref[0] = 0.0
        out_ref[1] = 0.0
        out_ref[2] = 0.0

    ones_col = jnp.ones((N, 1), jnp.float32)
    out_ref[0] += jnp.sum(jnp.dot(comp_f, ones_col,
                                  preferred_element_type=jnp.float32))
    out_ref[1] += jnp.sum(jnp.dot(con_f, ones_col,
                                  preferred_element_type=jnp.float32))
    out_ref[2] += jnp.sum(jnp.dot(tie_f, ones_col,
                                  preferred_element_type=jnp.float32))


@jax.jit
def _tc_counts(t, e, est):
    grid = (N - NSC) // TC_BR
    off = NSC // TC_BR
    row_spec = pl.BlockSpec((1, TC_BR), lambda i: (0, i + off))
    col_spec = pl.BlockSpec((1, N), lambda i: (0, 0))
    return pl.pallas_call(
        _tc_body,
        grid=(grid,),
        in_specs=[row_spec, row_spec, row_spec, col_spec, col_spec, col_spec],
        out_specs=pl.BlockSpec(memory_space=pltpu.SMEM),
        out_shape=jax.ShapeDtypeStruct((3,), jnp.float32),
    )(t, e, est, t, e, est)


def kernel(event_indicator, event_time, estimate):
    e = jnp.reshape(event_indicator, (-1,)).astype(jnp.int32)
    t = jnp.reshape(event_time, (-1,)).astype(jnp.float32)
    est = jnp.reshape(estimate, (-1,)).astype(jnp.float32)
    parts = _sc_counts(e, t, est)
    tc = _tc_counts(t.reshape(1, -1), e.reshape(1, -1), est.reshape(1, -1))
    total = jnp.sum(parts[:, 0, :]) + tc[0]
    con = jnp.sum(parts[:, 1, :]) + tc[1]
    tie = jnp.sum(parts[:, 2, :]) + tc[2]
    disc = total - con - tie
    loss = (disc + 0.5 * tie) / (disc + con + tie + 1e-07)
    return 1.0 - loss
